# Initial kernel scaffold; baseline (speedup 1.0000x reference)
import jax
import jax.numpy as jnp
from jax import lax
from jax.experimental import pallas as pl
from jax.experimental.pallas import tpu as pltpu
from jax.experimental.pallas import tpu_sc as plsc

N_NODES = 10000
N_EDGES = 320000
D_FEAT = 128
D_HID = 128
D_EDGE = 16

NC = 2
NS = 16
NW = NC * NS
G = 64
EPT = 10048
NG = EPT // G
N2 = 10240
ROWS_PER_TILE = N2 // NS
ACC_W = 144
BN = 512
BE = 2560

_f32 = jnp.float32
_i32 = jnp.int32


def _sc_mesh():
    return plsc.VectorSubcoreMesh(core_axis_name="c", subcore_axis_name="s",
                                  num_cores=NC, num_subcores=NS)


def _deg_body(ea_hbm, dst_hbm, acc_hbm, dst_v, ea_v, ext_v, acc_sh, sem):
    c = lax.axis_index("c")
    s = lax.axis_index("s")
    wid = s * NC + c

    pltpu.sync_copy(dst_hbm.at[wid], dst_v)

    for e in range(G):
        for j in range(32 // 16):
            ext_v[e, pl.ds(16 * j, 16)] = jnp.zeros((16,), _f32)
    for k in range(ROWS_PER_TILE // G):
        pltpu.sync_copy(ext_v, acc_sh.at[pl.ds(s * ROWS_PER_TILE + k * G, G)])
    plsc.subcore_barrier()

    ones = jnp.ones((16,), _f32)
    for e in range(G):
        ext_v[e, pl.ds(16, 16)] = ones

    def group(g, _):
        pltpu.sync_copy(ea_hbm.at[wid, g], ea_v)
        for e in range(G):
            ext_v[e, pl.ds(0, 16)] = ea_v[e, pl.ds(0, 16)]
        pltpu.sync_copy(ext_v, acc_sh.at[dst_v.at[g]], add=True)
        return 0

    lax.fori_loop(0, NG, group, 0)
    plsc.subcore_barrier()
    pltpu.sync_copy(acc_sh.at[pl.ds(s * ROWS_PER_TILE, ROWS_PER_TILE)],
                    acc_hbm.at[c, pl.ds(s * ROWS_PER_TILE, ROWS_PER_TILE)])


def _edge_body(src_hbm, dst_hbm, ae_hbm, asrc_hbm, adst_hbm, m_hbm, h_hbm,
               acc_hbm,
               src_v, dst_v, ae_v, asrc_v, adst_v, m_v, p_v, rows_v, ext_v,
               acc_sh, sem):
    c = lax.axis_index("c")
    s = lax.axis_index("s")
    wid = s * NC + c

    pltpu.sync_copy(src_hbm.at[wid], src_v)
    pltpu.sync_copy(dst_hbm.at[wid], dst_v)
    pltpu.sync_copy(ae_hbm.at[wid], ae_v)
    pltpu.sync_copy(asrc_hbm, asrc_v)
    pltpu.sync_copy(adst_hbm, adst_v)
    pltpu.sync_copy(m_hbm, m_v)

    for e in range(G):
        for j in range(ACC_W // 16):
            ext_v[e, pl.ds(16 * j, 16)] = jnp.zeros((16,), _f32)
    for k in range(ROWS_PER_TILE // G):
        pltpu.sync_copy(ext_v, acc_sh.at[pl.ds(s * ROWS_PER_TILE + k * G, G)])
    plsc.subcore_barrier()

    mvec = m_v[pl.ds(0, 16)]

    def group(g, _):
        cp = pltpu.async_copy(h_hbm.at[src_v.at[g]], rows_v, sem)
        for i in range(G // 16):
            sv = src_v[g, pl.ds(16 * i, 16)]
            dv = dst_v[g, pl.ds(16 * i, 16)]
            sc = (plsc.load_gather(asrc_v, [sv])
                  + plsc.load_gather(adst_v, [dv])
                  + ae_v[g, pl.ds(16 * i, 16)])
            al = jnp.where(sc >= 0.0, sc, 0.2 * sc)
            p_v[pl.ds(16 * i, 16)] = jnp.exp(al - mvec)
        cp.wait()
        for e in range(G):
            p16 = plsc.load_gather(p_v, [jnp.full((16,), e, _i32)])
            for j in range(D_HID // 16):
                ext_v[e, pl.ds(16 * j, 16)] = rows_v[e, pl.ds(16 * j, 16)] * p16
            ext_v[e, pl.ds(D_HID, 16)] = p16
        pltpu.sync_copy(ext_v, acc_sh.at[dst_v.at[g]], add=True)
        return 0

    lax.fori_loop(0, NG, group, 0)
    plsc.subcore_barrier()
    pltpu.sync_copy(acc_sh.at[pl.ds(s * ROWS_PER_TILE, ROWS_PER_TILE)],
                    acc_hbm.at[c, pl.ds(s * ROWS_PER_TILE, ROWS_PER_TILE)])


def _sc_deg(ea_t, dst_t):
    fn = pl.kernel(
        _deg_body,
        out_type=jax.ShapeDtypeStruct((NC, N2, 32), _f32),
        mesh=_sc_mesh(),
        scratch_types=[
            pltpu.VMEM((NG, G), _i32),
            pltpu.VMEM((G, D_EDGE), _f32),
            pltpu.VMEM((G, 32), _f32),
            pltpu.VMEM_SHARED((N2, 32), _f32),
            pltpu.SemaphoreType.DMA,
        ],
    )
    return fn(ea_t, dst_t)


def _sc_edges(src_t, dst_t, ae_t, asrc, adst, m_arr, h):
    fn = pl.kernel(
        _edge_body,
        out_type=jax.ShapeDtypeStruct((NC, N2, ACC_W), _f32),
        mesh=_sc_mesh(),
        scratch_types=[
            pltpu.VMEM((NG, G), _i32),
            pltpu.VMEM((NG, G), _i32),
            pltpu.VMEM((NG, G), _f32),
            pltpu.VMEM((N2,), _f32),
            pltpu.VMEM((N2,), _f32),
            pltpu.VMEM((16,), _f32),
            pltpu.VMEM((G,), _f32),
            pltpu.VMEM((G, D_HID), _f32),
            pltpu.VMEM((G, ACC_W), _f32),
            pltpu.VMEM_SHARED((N2, ACC_W), _f32),
            pltpu.SemaphoreType.DMA,
        ],
    )
    return fn(src_t, dst_t, ae_t, asrc, adst, m_arr, h)


def _ae_body(ea_ref, vem_ref, ae1_ref, ae2_ref):
    ea = ea_ref[...]
    ae1_ref[...] = (ea * vem_ref[0:1, 0:D_EDGE]).sum(-1)
    ae2_ref[...] = (ea * vem_ref[1:2, 0:D_EDGE]).sum(-1)


def _tc_ae(edge_attr, vem):
    grid = N_EDGES // BE
    return pl.pallas_call(
        _ae_body,
        grid=(grid,),
        in_specs=[
            pl.BlockSpec((BE, D_EDGE), lambda i: (i, 0)),
            pl.BlockSpec((8, 128), lambda i: (0, 0)),
        ],
        out_specs=[
            pl.BlockSpec((BE,), lambda i: (i,)),
            pl.BlockSpec((BE,), lambda i: (i,)),
        ],
        out_shape=[
            jax.ShapeDtypeStruct((N_EDGES,), _f32),
            jax.ShapeDtypeStruct((N_EDGES,), _f32),
        ],
    )(edge_attr, vem)


def _n1_body(x_ref, w1t_ref, attm_ref, vem_ref, acc0_ref,
             h1_ref, asrc_ref, adst_ref, aeL1_ref, aeL2_ref):
    h = jnp.dot(x_ref[...], w1t_ref[...], preferred_element_type=_f32)
    h1_ref[...] = h
    asrc_ref[...] = (h * attm_ref[0:1, :]).sum(-1)
    adst_ref[...] = (h * attm_ref[1:2, :]).sum(-1)
    a0 = acc0_ref[0] + acc0_ref[1]
    deg = a0[:, D_EDGE:D_EDGE + 1]
    la = a0[:, 0:D_EDGE] / jnp.maximum(deg, 1.0)
    aeL1_ref[...] = (la * vem_ref[0:1, 0:D_EDGE]).sum(-1)
    aeL2_ref[...] = (la * vem_ref[1:2, 0:D_EDGE]).sum(-1)


def _tc_n1(x2, w1t, attm, vem, acc0):
    grid = N2 // BN
    vec = jax.ShapeDtypeStruct((N2,), _f32)
    vspec = pl.BlockSpec((BN,), lambda i: (i,))
    return pl.pallas_call(
        _n1_body,
        grid=(grid,),
        in_specs=[
            pl.BlockSpec((BN, D_FEAT), lambda i: (i, 0)),
            pl.BlockSpec((D_FEAT, D_HID), lambda i: (0, 0)),
            pl.BlockSpec((8, 128), lambda i: (0, 0)),
            pl.BlockSpec((8, 128), lambda i: (0, 0)),
            pl.BlockSpec((NC, BN, 32), lambda i: (0, i, 0)),
        ],
        out_specs=[
            pl.BlockSpec((BN, D_HID), lambda i: (i, 0)),
            vspec, vspec, vspec, vspec,
        ],
        out_shape=[jax.ShapeDtypeStruct((N2, D_HID), _f32), vec, vec, vec, vec],
    )(x2, w1t, attm, vem, acc0)


def _n2_body(h1_ref, asrc_ref, adst_ref, aeL_ref, mb_ref, acc_ref, bw_ref,
             w2t_ref, attm_ref, h2_ref, asrc2_ref, adst2_ref):
    m = mb_ref[0, 0:1]
    s = asrc_ref[...] + adst_ref[...] + aeL_ref[...]
    al = jnp.where(s >= 0.0, s, 0.2 * s)
    ps = jnp.exp(al - m)
    acc = acc_ref[0] + acc_ref[1]
    num = acc[:, 0:D_HID] + ps[:, None] * h1_ref[...]
    ssum = acc[:, D_HID] + ps
    o1 = num / (ssum + 1e-16)[:, None] + bw_ref[0:1, :]
    h1r = jnp.maximum(o1, 0.0)
    h2 = jnp.dot(h1r, w2t_ref[...], preferred_element_type=_f32)
    h2_ref[...] = h2
    asrc2_ref[...] = (h2 * attm_ref[0:1, :]).sum(-1)
    adst2_ref[...] = (h2 * attm_ref[1:2, :]).sum(-1)


def _tc_n2(h1, asrc1, adst1, aeL1, mb1, acc1, bw1, w2t, attm2):
    grid = N2 // BN
    vec = jax.ShapeDtypeStruct((N2,), _f32)
    vspec = pl.BlockSpec((BN,), lambda i: (i,))
    return pl.pallas_call(
        _n2_body,
        grid=(grid,),
        in_specs=[
            pl.BlockSpec((BN, D_HID), lambda i: (i, 0)),
            vspec, vspec, vspec,
            pl.BlockSpec((8, 128), lambda i: (0, 0)),
            pl.BlockSpec((NC, BN, ACC_W), lambda i: (0, i, 0)),
            pl.BlockSpec((8, 128), lambda i: (0, 0)),
            pl.BlockSpec((D_HID, D_HID), lambda i: (0, 0)),
            pl.BlockSpec((8, 128), lambda i: (0, 0)),
        ],
        out_specs=[pl.BlockSpec((BN, D_HID), lambda i: (i, 0)), vspec, vspec],
        out_shape=[jax.ShapeDtypeStruct((N2, D_HID), _f32), vec, vec],
    )(h1, asrc1, adst1, aeL1, mb1, acc1, bw1, w2t, attm2)


def _n3_body(h2_ref, asrc_ref, adst_ref, aeL_ref, mb_ref, acc_ref, bw_ref,
             lin_ref, y_ref):
    m = mb_ref[0, 0:1]
    s = asrc_ref[...] + adst_ref[...] + aeL_ref[...]
    al = jnp.where(s >= 0.0, s, 0.2 * s)
    ps = jnp.exp(al - m)
    acc = acc_ref[0] + acc_ref[1]
    num = acc[:, 0:D_HID] + ps[:, None] * h2_ref[...]
    ssum = acc[:, D_HID] + ps
    o2 = num / (ssum + 1e-16)[:, None] + bw_ref[0:1, :]
    y = (o2 * lin_ref[0:1, :]).sum(-1) + lin_ref[1, 0:1]
    y_ref[...] = jnp.maximum(y, 0.0)


def _tc_n3(h2, asrc2, adst2, aeL2, mb2, acc2, bw2, linm):
    grid = N2 // BN
    vspec = pl.BlockSpec((BN,), lambda i: (i,))
    return pl.pallas_call(
        _n3_body,
        grid=(grid,),
        in_specs=[
            pl.BlockSpec((BN, D_HID), lambda i: (i, 0)),
            vspec, vspec, vspec,
            pl.BlockSpec((8, 128), lambda i: (0, 0)),
            pl.BlockSpec((NC, BN, ACC_W), lambda i: (0, i, 0)),
            pl.BlockSpec((8, 128), lambda i: (0, 0)),
            pl.BlockSpec((8, 128), lambda i: (0, 0)),
        ],
        out_specs=vspec,
        out_shape=jax.ShapeDtypeStruct((N2,), _f32),
    )(h2, asrc2, adst2, aeL2, mb2, acc2, bw2, linm)


def _pad_rows8(v):
    out = jnp.zeros((8, 128), _f32)
    if v.ndim == 1:
        return out.at[0, :v.shape[0]].set(v)
    return out.at[:v.shape[0], :v.shape[1]].set(v)


def _tile_edges(v, pad_val):
    v = v.reshape(NW, N_EDGES // NW)
    pad = jnp.broadcast_to(pad_val, (NW, EPT - N_EDGES // NW)).astype(v.dtype)
    return jnp.concatenate([v, pad], axis=1).reshape(NW, NG, G)


def _lrelu_scalar(x):
    return jnp.where(x >= 0.0, x, 0.2 * x)


@jax.jit
def kernel(x, edge_index, edge_attr, W1, att_src1, att_dst1, We1, att_e1, b1,
           W2, att_src2, att_dst2, We2, att_e2, b2, linW, linb):
    src = edge_index[0].astype(_i32)
    dst = edge_index[1].astype(_i32)

    ve1 = We1.T @ att_e1
    ve2 = We2.T @ att_e2
    vem = _pad_rows8(jnp.stack([ve1, ve2]))
    attm1 = _pad_rows8(jnp.stack([att_src1, att_dst1]))
    attm2 = _pad_rows8(jnp.stack([att_src2, att_dst2]))
    bw1 = _pad_rows8(b1)
    bw2 = _pad_rows8(b2)
    linm = _pad_rows8(linW[0]).at[1, 0].set(linb[0])
    x2 = jnp.zeros((N2, D_FEAT), _f32).at[:N_NODES].set(x)

    src_t = _tile_edges(src, 0)
    sent = N_NODES + (jnp.arange(EPT - N_EDGES // NW, dtype=_i32) % 16)
    dst_t = _tile_edges(dst, sent)
    ea_t = jnp.concatenate(
        [edge_attr.reshape(NW, N_EDGES // NW, D_EDGE),
         jnp.zeros((NW, EPT - N_EDGES // NW, D_EDGE), _f32)],
        axis=1).reshape(NW, NG, G, D_EDGE)

    acc0 = _sc_deg(ea_t, dst_t)

    ae1, ae2 = _tc_ae(edge_attr, vem)

    h1, asrc1, adst1, aeL1, aeL2 = _tc_n1(x2, W1.T, attm1, vem, acc0)

    m1 = _lrelu_scalar(
        jnp.max(asrc1[:N_NODES]) + jnp.max(adst1[:N_NODES])
        + jnp.maximum(jnp.max(ae1), jnp.max(aeL1[:N_NODES])))
    m1_arr = jnp.full((16,), m1, _f32)
    mb1 = jnp.full((8, 128), m1, _f32)

    ae1_t = _tile_edges(ae1, jnp.float32(-1e30))
    ae2_t = _tile_edges(ae2, jnp.float32(-1e30))

    acc1 = _sc_edges(src_t, dst_t, ae1_t, asrc1, adst1, m1_arr, h1)

    h2, asrc2, adst2 = _tc_n2(h1, asrc1, adst1, aeL1, mb1, acc1, bw1,
                              W2.T, attm2)

    m2 = _lrelu_scalar(
        jnp.max(asrc2[:N_NODES]) + jnp.max(adst2[:N_NODES])
        + jnp.maximum(jnp.max(ae2), jnp.max(aeL2[:N_NODES])))
    m2_arr = jnp.full((16,), m2, _f32)
    mb2 = jnp.full((8, 128), m2, _f32)

    acc2 = _sc_edges(src_t, dst_t, ae2_t, asrc2, adst2, m2_arr, h2)

    y = _tc_n3(h2, asrc2, adst2, aeL2, mb2, acc2, bw2, linm)
    return y[:N_NODES].reshape(N_NODES, 1)


# speedup vs baseline: 11.4696x; 11.4696x over previous
import functools

import jax
import jax.numpy as jnp
from jax import lax
from jax.experimental import pallas as pl
from jax.experimental.pallas import tpu as pltpu
from jax.experimental.pallas import tpu_sc as plsc

N_NODES = 10000
N_EDGES = 320000
D_FEAT = 128
D_HID = 128
D_EDGE = 16

NC = 2
NS = 16
NW = NC * NS
G = 64
EPT = 10240
NG = EPT // G
WG = 16
NWIN = NG // WG
N2 = 10240
ROWS_PER_TILE = N2 // NS
BN = 512
BE = 4096
E2 = 327680

_f32 = jnp.float32
_i32 = jnp.int32


def _sc_mesh():
    return plsc.VectorSubcoreMesh(core_axis_name="c", subcore_axis_name="s",
                                  num_cores=NC, num_subcores=NS)


_SC_PARAMS = pltpu.CompilerParams(needs_layout_passes=False)


def _splat(vec, lane):
    return jnp.take_along_axis(vec, jnp.full((16,), lane, _i32), axis=0,
                               mode="promise_in_bounds")


def _score_body(nstat,
                src_hbm, dst_hbm, ae_hbm, asrc_hbm, adst_hbm,
                s_hbm, stat_hbm,
                src_w, dst_w, ae_w, asrc_v, adst_v, s_w, stat_v):
    c = lax.axis_index("c")
    s = lax.axis_index("s")
    wid = s * NC + c

    pltpu.sync_copy(asrc_hbm, asrc_v)
    pltpu.sync_copy(adst_hbm, adst_v)

    zero16 = jnp.zeros((16,), _f32)

    def zstat(i, _):
        stat_v[pl.ds(i * 16, 16)] = zero16
        return 0
    lax.fori_loop(0, (N2 * nstat) // 16, zstat, 0)

    iota = lax.iota(_i32, 16)
    ones = jnp.ones((16,), _f32)
    stat_mask = iota < nstat

    def group(gg, _):
        for i in range(G // 16):
            sv = src_w[gg, pl.ds(16 * i, 16)]
            dv = dst_w[gg, pl.ds(16 * i, 16)]
            aev = ae_w[gg, pl.ds(16 * i, 16)]
            s_w[gg, pl.ds(16 * i, 16)] = (
                plsc.load_gather(asrc_v, [sv])
                + plsc.load_gather(adst_v, [dv])
                + aev)
            for l in range(16):
                idxs = _splat(dv, l) * nstat + iota
                if nstat == 2:
                    val = jnp.where(iota == 0, _splat(aev, l), ones)
                else:
                    val = _splat(aev, l)
                plsc.addupdate_scatter(stat_v, [idxs], val, mask=stat_mask)
        return 0

    def window(w, _):
        pltpu.sync_copy(src_hbm.at[wid, pl.ds(w * WG, WG)], src_w)
        pltpu.sync_copy(dst_hbm.at[wid, pl.ds(w * WG, WG)], dst_w)
        pltpu.sync_copy(ae_hbm.at[wid, pl.ds(w * WG, WG)], ae_w)
        lax.fori_loop(0, WG, group, 0)
        pltpu.sync_copy(s_w, s_hbm.at[wid, pl.ds(w * WG, WG)])
        return 0

    lax.fori_loop(0, NWIN, window, 0)
    pltpu.sync_copy(stat_v, stat_hbm.at[wid])


def _sc_scores(src_t, dst_t, ae_t, asrc, adst, nstat):
    fn = pl.kernel(
        functools.partial(_score_body, nstat),
        out_type=(
            jax.ShapeDtypeStruct((NW, NG, G), _f32),
            jax.ShapeDtypeStruct((NW, N2 * nstat), _f32),
        ),
        mesh=_sc_mesh(),
        scratch_types=[
            pltpu.VMEM((WG, G), _i32),
            pltpu.VMEM((WG, G), _i32),
            pltpu.VMEM((WG, G), _f32),
            pltpu.VMEM((N2,), _f32),
            pltpu.VMEM((N2,), _f32),
            pltpu.VMEM((WG, G), _f32),
            pltpu.VMEM((N2 * nstat,), _f32),
        ],
        compiler_params=_SC_PARAMS,
    )
    return fn(src_t, dst_t, ae_t, asrc, adst)


def _scatter_body(src_hbm, dst_hbm, p_hbm, h_hbm, acc_hbm, psum_hbm,
                  src_w, dst_w, p_w, rows_v, psum_v, acc_sh, sem):
    c = lax.axis_index("c")
    s = lax.axis_index("s")
    wid = s * NC + c

    zero16 = jnp.zeros((16,), _f32)

    def zps(i, _):
        psum_v[pl.ds(i * 16, 16)] = zero16
        return 0
    lax.fori_loop(0, N2 // 16, zps, 0)

    for e in range(G):
        for j in range(D_HID // 16):
            rows_v[e, pl.ds(16 * j, 16)] = zero16
    for k in range(ROWS_PER_TILE // G):
        pltpu.sync_copy(rows_v, acc_sh.at[pl.ds(s * ROWS_PER_TILE + k * G, G)])
    plsc.subcore_barrier()

    iota = lax.iota(_i32, 16)
    lane0 = iota == 0

    def group(gg, _):
        cp = pltpu.async_copy(h_hbm.at[src_w.at[gg]], rows_v, sem)
        cp.wait()
        for i in range(G // 16):
            pvec = p_w[gg, pl.ds(16 * i, 16)]
            dvec = dst_w[gg, pl.ds(16 * i, 16)]
            for l in range(16):
                e = i * 16 + l
                p16 = _splat(pvec, l)
                for j in range(D_HID // 16):
                    rows_v[e, pl.ds(16 * j, 16)] = (
                        rows_v[e, pl.ds(16 * j, 16)] * p16)
                plsc.addupdate_scatter(psum_v, [_splat(dvec, l) + iota], p16,
                                       mask=lane0)
        pltpu.sync_copy(rows_v, acc_sh.at[dst_w.at[gg]], add=True)
        return 0

    def window(w, _):
        pltpu.sync_copy(src_hbm.at[wid, pl.ds(w * WG, WG)], src_w)
        pltpu.sync_copy(dst_hbm.at[wid, pl.ds(w * WG, WG)], dst_w)
        pltpu.sync_copy(p_hbm.at[wid, pl.ds(w * WG, WG)], p_w)
        lax.fori_loop(0, WG, group, 0)
        return 0

    lax.fori_loop(0, NWIN, window, 0)
    pltpu.sync_copy(psum_v, psum_hbm.at[wid])
    plsc.subcore_barrier()
    pltpu.sync_copy(acc_sh.at[pl.ds(s * ROWS_PER_TILE, ROWS_PER_TILE)],
                    acc_hbm.at[c, pl.ds(s * ROWS_PER_TILE, ROWS_PER_TILE)])


def _sc_scatter(src_t, dst_t, p_t, h):
    fn = pl.kernel(
        _scatter_body,
        out_type=(
            jax.ShapeDtypeStruct((NC, N2, D_HID), _f32),
            jax.ShapeDtypeStruct((NW, N2), _f32),
        ),
        mesh=_sc_mesh(),
        scratch_types=[
            pltpu.VMEM((WG, G), _i32),
            pltpu.VMEM((WG, G), _i32),
            pltpu.VMEM((WG, G), _f32),
            pltpu.VMEM((G, D_HID), _f32),
            pltpu.VMEM((N2,), _f32),
            pltpu.VMEM_SHARED((N2, D_HID), _f32),
            pltpu.SemaphoreType.DMA,
        ],
        compiler_params=_SC_PARAMS,
    )
    return fn(src_t, dst_t, p_t, h)


def _ae_body(ea_ref, vem_ref, ae1_ref, ae2_ref):
    ea = ea_ref[...].astype(jnp.bfloat16).astype(_f32)
    ae1_ref[...] = (ea * vem_ref[0:1, 0:D_EDGE]).sum(-1)
    ae2_ref[...] = (ea * vem_ref[1:2, 0:D_EDGE]).sum(-1)


def _tc_ae(edge_attr_pad, vem):
    grid = E2 // BE
    return pl.pallas_call(
        _ae_body,
        grid=(grid,),
        in_specs=[
            pl.BlockSpec((BE, D_EDGE), lambda i: (i, 0)),
            pl.BlockSpec((8, 128), lambda i: (0, 0)),
        ],
        out_specs=[
            pl.BlockSpec((BE,), lambda i: (i,)),
            pl.BlockSpec((BE,), lambda i: (i,)),
        ],
        out_shape=[
            jax.ShapeDtypeStruct((E2,), _f32),
            jax.ShapeDtypeStruct((E2,), _f32),
        ],
    )(edge_attr_pad, vem)


def _exp_body(s_ref, mb_ref, p_ref):
    s = s_ref[...]
    al = jnp.where(s >= 0.0, s, 0.2 * s)
    p_ref[...] = jnp.exp(al - mb_ref[0, 0:1])


def _tc_exp(s_flat, mb):
    grid = E2 // BE
    return pl.pallas_call(
        _exp_body,
        grid=(grid,),
        in_specs=[
            pl.BlockSpec((BE,), lambda i: (i,)),
            pl.BlockSpec((8, 128), lambda i: (0, 0)),
        ],
        out_specs=pl.BlockSpec((BE,), lambda i: (i,)),
        out_shape=jax.ShapeDtypeStruct((E2,), _f32),
    )(s_flat, mb)


def _n1_body(x_ref, w1t_ref, attm_ref, h1_ref, asrc_ref, adst_ref):
    h = jnp.dot(x_ref[...], w1t_ref[...], preferred_element_type=_f32)
    h1_ref[...] = h
    asrc_ref[...] = (h * attm_ref[0:1, :]).sum(-1)
    adst_ref[...] = (h * attm_ref[1:2, :]).sum(-1)


def _tc_n1(x2, w1t, attm):
    grid = N2 // BN
    vec = jax.ShapeDtypeStruct((N2,), _f32)
    vspec = pl.BlockSpec((BN,), lambda i: (i,))
    return pl.pallas_call(
        _n1_body,
        grid=(grid,),
        in_specs=[
            pl.BlockSpec((BN, D_FEAT), lambda i: (i, 0)),
            pl.BlockSpec((D_FEAT, D_HID), lambda i: (0, 0)),
            pl.BlockSpec((8, 128), lambda i: (0, 0)),
        ],
        out_specs=[pl.BlockSpec((BN, D_HID), lambda i: (i, 0)), vspec, vspec],
        out_shape=[jax.ShapeDtypeStruct((N2, D_HID), _f32), vec, vec],
    )(x2, w1t, attm)


def _n2_body(h1_ref, asrc_ref, adst_ref, mb_ref, acc_ref, stat_ref, psum_ref,
             bw_ref, w2t_ref, attm_ref, h2_ref, asrc2_ref, adst2_ref,
             degc_ref):
    m = mb_ref[0, 0:1]
    st = stat_ref[...].sum(0)
    degc = jnp.maximum(st[:, 1], 1.0)
    degc_ref[...] = degc
    aeL1 = st[:, 0] / degc
    s = asrc_ref[...] + adst_ref[...] + aeL1
    al = jnp.where(s >= 0.0, s, 0.2 * s)
    ps = jnp.exp(al - m)
    num = acc_ref[0] + acc_ref[1] + ps[:, None] * h1_ref[...]
    ssum = psum_ref[...].sum(0) + ps
    o1 = num / (ssum + 1e-16)[:, None] + bw_ref[0:1, :]
    h1r = jnp.maximum(o1, 0.0)
    h2 = jnp.dot(h1r, w2t_ref[...], preferred_element_type=_f32)
    h2_ref[...] = h2
    asrc2_ref[...] = (h2 * attm_ref[0:1, :]).sum(-1)
    adst2_ref[...] = (h2 * attm_ref[1:2, :]).sum(-1)


def _tc_n2(h1, asrc1, adst1, mb1, acc1, stat1, psum1, bw1, w2t, attm2):
    grid = N2 // BN
    vec = jax.ShapeDtypeStruct((N2,), _f32)
    vspec = pl.BlockSpec((BN,), lambda i: (i,))
    return pl.pallas_call(
        _n2_body,
        grid=(grid,),
        in_specs=[
            pl.BlockSpec((BN, D_HID), lambda i: (i, 0)),
            vspec, vspec,
            pl.BlockSpec((8, 128), lambda i: (0, 0)),
            pl.BlockSpec((NC, BN, D_HID), lambda i: (0, i, 0)),
            pl.BlockSpec((NW, BN, 2), lambda i: (0, i, 0)),
            pl.BlockSpec((NW, BN), lambda i: (0, i)),
            pl.BlockSpec((8, 128), lambda i: (0, 0)),
            pl.BlockSpec((D_HID, D_HID), lambda i: (0, 0)),
            pl.BlockSpec((8, 128), lambda i: (0, 0)),
        ],
        out_specs=[
            pl.BlockSpec((BN, D_HID), lambda i: (i, 0)),
            vspec, vspec, vspec,
        ],
        out_shape=[jax.ShapeDtypeStruct((N2, D_HID), _f32), vec, vec, vec],
    )(h1, asrc1, adst1, mb1, acc1, stat1, psum1, bw1, w2t, attm2)


def _n3_body(h2_ref, asrc_ref, adst_ref, degc_ref, mb_ref, acc_ref, stat_ref,
             psum_ref, bw_ref, lin_ref, y_ref):
    m = mb_ref[0, 0:1]
    aeL2 = stat_ref[...].sum(0) / degc_ref[...]
    s = asrc_ref[...] + adst_ref[...] + aeL2
    al = jnp.where(s >= 0.0, s, 0.2 * s)
    ps = jnp.exp(al - m)
    num = acc_ref[0] + acc_ref[1] + ps[:, None] * h2_ref[...]
    ssum = psum_ref[...].sum(0) + ps
    o2 = num / (ssum + 1e-16)[:, None] + bw_ref[0:1, :]
    y = (o2 * lin_ref[0:1, :]).sum(-1) + lin_ref[1, 0:1]
    y_ref[...] = jnp.maximum(y, 0.0)


def _tc_n3(h2, asrc2, adst2, degc, mb2, acc2, stat2, psum2, bw2, linm):
    grid = N2 // BN
    vspec = pl.BlockSpec((BN,), lambda i: (i,))
    return pl.pallas_call(
        _n3_body,
        grid=(grid,),
        in_specs=[
            pl.BlockSpec((BN, D_HID), lambda i: (i, 0)),
            vspec, vspec, vspec,
            pl.BlockSpec((8, 128), lambda i: (0, 0)),
            pl.BlockSpec((NC, BN, D_HID), lambda i: (0, i, 0)),
            pl.BlockSpec((NW, BN), lambda i: (0, i)),
            pl.BlockSpec((NW, BN), lambda i: (0, i)),
            pl.BlockSpec((8, 128), lambda i: (0, 0)),
            pl.BlockSpec((8, 128), lambda i: (0, 0)),
        ],
        out_specs=vspec,
        out_shape=jax.ShapeDtypeStruct((N2,), _f32),
    )(h2, asrc2, adst2, degc, mb2, acc2, stat2, psum2, bw2, linm)


def _pad_rows8(v):
    out = jnp.zeros((8, 128), _f32)
    if v.ndim == 1:
        return out.at[0, :v.shape[0]].set(v)
    return out.at[:v.shape[0], :v.shape[1]].set(v)


def _tile_edges(v, pad_val):
    v = v.reshape(NW, N_EDGES // NW)
    pad = jnp.broadcast_to(pad_val, (NW, EPT - N_EDGES // NW)).astype(v.dtype)
    return jnp.concatenate([v, pad], axis=1).reshape(NW, NG, G)


def _lrelu_scalar(x):
    return jnp.where(x >= 0.0, x, 0.2 * x)


@jax.jit
def kernel(x, edge_index, edge_attr, W1, att_src1, att_dst1, We1, att_e1, b1,
           W2, att_src2, att_dst2, We2, att_e2, b2, linW, linb):
    src = edge_index[0].astype(_i32)
    dst = edge_index[1].astype(_i32)

    ve1 = We1.astype(jnp.bfloat16).astype(_f32).T @ att_e1
    ve2 = We2.astype(jnp.bfloat16).astype(_f32).T @ att_e2
    vem = _pad_rows8(jnp.stack([ve1, ve2]))
    attm1 = _pad_rows8(jnp.stack([att_src1, att_dst1]))
    attm2 = _pad_rows8(jnp.stack([att_src2, att_dst2]))
    bw1 = _pad_rows8(b1)
    bw2 = _pad_rows8(b2)
    linm = _pad_rows8(linW[0]).at[1, 0].set(linb[0])
    x2 = jnp.zeros((N2, D_FEAT), _f32).at[:N_NODES].set(x)

    src_t = _tile_edges(src, 0)
    sent = N_NODES + (jnp.arange(EPT - N_EDGES // NW, dtype=_i32) % 16)
    dst_t = _tile_edges(dst, sent)

    ea_pad = jnp.zeros((E2, D_EDGE), _f32).at[:N_EDGES].set(edge_attr)
    ae1, ae2 = _tc_ae(ea_pad, vem)
    ae1 = ae1[:N_EDGES]
    ae2 = ae2[:N_EDGES]
    ae1_t = _tile_edges(ae1, jnp.float32(-1e30))

    h1, asrc1, adst1 = _tc_n1(x2, W1.T, attm1)

    m1 = _lrelu_scalar(
        jnp.max(asrc1[:N_NODES]) + jnp.max(adst1[:N_NODES])
        + jnp.maximum(jnp.max(ae1), 0.0))
    mb1 = jnp.full((8, 128), m1, _f32)

    s1_t, stat1 = _sc_scores(src_t, dst_t, ae1_t, asrc1, adst1, 2)
    stat1 = stat1.reshape(NW, N2, 2)
    p1_t = _tc_exp(s1_t.reshape(E2), mb1).reshape(NW, NG, G)
    acc1, psum1 = _sc_scatter(src_t, dst_t, p1_t, h1)

    h2, asrc2, adst2, degc = _tc_n2(h1, asrc1, adst1, mb1, acc1, stat1,
                                    psum1, bw1, W2.T, attm2)

    m2 = _lrelu_scalar(
        jnp.max(asrc2[:N_NODES]) + jnp.max(adst2[:N_NODES])
        + jnp.maximum(jnp.max(ae2), 0.0))
    mb2 = jnp.full((8, 128), m2, _f32)

    ae2s_t = _tile_edges(ae2, jnp.float32(-1e30))
    s2_t, stat2 = _sc_scores(src_t, dst_t, ae2s_t, asrc2, adst2, 1)
    stat2 = stat2.reshape(NW, N2)
    p2_t = _tc_exp(s2_t.reshape(E2), mb2).reshape(NW, NG, G)
    acc2, psum2 = _sc_scatter(src_t, dst_t, p2_t, h2)

    y = _tc_n3(h2, asrc2, adst2, degc, mb2, acc2, stat2, psum2, bw2, linm)
    return y[:N_NODES].reshape(N_NODES, 1)


# speedup vs baseline: 11.9698x; 1.0436x over previous
import functools

import jax
import jax.numpy as jnp
from jax import lax
from jax.experimental import pallas as pl
from jax.experimental.pallas import tpu as pltpu
from jax.experimental.pallas import tpu_sc as plsc

N_NODES = 10000
N_EDGES = 320000
D_FEAT = 128
D_HID = 128
D_EDGE = 16

NC = 2
NS = 16
NW = NC * NS
G = 64
EPT = 10240
NG = EPT // G
WG = 16
NWIN = NG // WG
N2 = 10240
ROWS_PER_TILE = N2 // NS
BN = 512
BE = 4096
E2 = 327680

_f32 = jnp.float32
_i32 = jnp.int32


def _sc_mesh():
    return plsc.VectorSubcoreMesh(core_axis_name="c", subcore_axis_name="s",
                                  num_cores=NC, num_subcores=NS)


_SC_PARAMS = pltpu.CompilerParams(needs_layout_passes=False)


def _splat(vec, lane):
    return jnp.take_along_axis(vec, jnp.full((16,), lane, _i32), axis=0,
                               mode="promise_in_bounds")


def _score_body(nstat,
                src_hbm, dst_hbm, ae_hbm, asrc_hbm, adst_hbm,
                s_hbm, stat_hbm,
                src_w, dst_w, ae_w, asrc_v, adst_v, s_w, stat_v):
    c = lax.axis_index("c")
    s = lax.axis_index("s")
    wid = s * NC + c

    pltpu.sync_copy(asrc_hbm, asrc_v)
    pltpu.sync_copy(adst_hbm, adst_v)

    zero16 = jnp.zeros((16,), _f32)

    def zstat(i, _):
        stat_v[pl.ds(i * 16, 16)] = zero16
        return 0
    lax.fori_loop(0, (N2 * nstat) // 16, zstat, 0)

    iota = lax.iota(_i32, 16)
    ones = jnp.ones((16,), _f32)
    stat_mask = iota < nstat

    def group(gg, _):
        for i in range(G // 16):
            sv = src_w[gg, pl.ds(16 * i, 16)]
            dv = dst_w[gg, pl.ds(16 * i, 16)]
            aev = ae_w[gg, pl.ds(16 * i, 16)]
            s_w[gg, pl.ds(16 * i, 16)] = (
                plsc.load_gather(asrc_v, [sv])
                + plsc.load_gather(adst_v, [dv])
                + aev)
            for l in range(16):
                idxs = _splat(dv, l) * nstat + iota
                if nstat == 2:
                    val = jnp.where(iota == 0, _splat(aev, l), ones)
                else:
                    val = _splat(aev, l)
                plsc.addupdate_scatter(stat_v, [idxs], val, mask=stat_mask)
        return 0

    def window(w, _):
        pltpu.sync_copy(src_hbm.at[wid, pl.ds(w * WG, WG)], src_w)
        pltpu.sync_copy(dst_hbm.at[wid, pl.ds(w * WG, WG)], dst_w)
        pltpu.sync_copy(ae_hbm.at[wid, pl.ds(w * WG, WG)], ae_w)
        lax.fori_loop(0, WG, group, 0)
        pltpu.sync_copy(s_w, s_hbm.at[wid, pl.ds(w * WG, WG)])
        return 0

    lax.fori_loop(0, NWIN, window, 0)
    pltpu.sync_copy(stat_v, stat_hbm.at[wid])


def _sc_scores(src_t, dst_t, ae_t, asrc, adst, nstat):
    fn = pl.kernel(
        functools.partial(_score_body, nstat),
        out_type=(
            jax.ShapeDtypeStruct((NW, NG, G), _f32),
            jax.ShapeDtypeStruct((NW, N2 * nstat), _f32),
        ),
        mesh=_sc_mesh(),
        scratch_types=[
            pltpu.VMEM((WG, G), _i32),
            pltpu.VMEM((WG, G), _i32),
            pltpu.VMEM((WG, G), _f32),
            pltpu.VMEM((N2,), _f32),
            pltpu.VMEM((N2,), _f32),
            pltpu.VMEM((WG, G), _f32),
            pltpu.VMEM((N2 * nstat,), _f32),
        ],
        compiler_params=_SC_PARAMS,
    )
    return fn(src_t, dst_t, ae_t, asrc, adst)


def _scatter_body(src_hbm, dst_hbm, p_hbm, h_hbm, acc_hbm, psum_hbm,
                  src_w, dst_w, p_w, rows_a, rows_b, ext_v, psum_v, acc_sh,
                  sem_a, sem_b):
    c = lax.axis_index("c")
    s = lax.axis_index("s")
    wid = s * NC + c

    zero16 = jnp.zeros((16,), _f32)

    def zps(i, _):
        psum_v[pl.ds(i * 16, 16)] = zero16
        return 0
    lax.fori_loop(0, N2 // 16, zps, 0)

    for e in range(G):
        for j in range(D_HID // 16):
            ext_v[e, pl.ds(16 * j, 16)] = zero16
    for k in range(ROWS_PER_TILE // G):
        pltpu.sync_copy(ext_v, acc_sh.at[pl.ds(s * ROWS_PER_TILE + k * G, G)])
    plsc.subcore_barrier()

    iota = lax.iota(_i32, 16)
    lane0 = iota == 0

    def scale_scatter(gg, rows):
        for i in range(G // 16):
            pvec = p_w[gg, pl.ds(16 * i, 16)]
            dvec = dst_w[gg, pl.ds(16 * i, 16)]
            for l in range(16):
                e = i * 16 + l
                p16 = _splat(pvec, l)
                for j in range(D_HID // 16):
                    ext_v[e, pl.ds(16 * j, 16)] = (
                        rows[e, pl.ds(16 * j, 16)] * p16)
                plsc.addupdate_scatter(psum_v, [_splat(dvec, l) + iota], p16,
                                       mask=lane0)
        pltpu.sync_copy(ext_v, acc_sh.at[dst_w.at[gg]], add=True)

    def window(w, _):
        pltpu.sync_copy(src_hbm.at[wid, pl.ds(w * WG, WG)], src_w)
        pltpu.sync_copy(dst_hbm.at[wid, pl.ds(w * WG, WG)], dst_w)
        pltpu.sync_copy(p_hbm.at[wid, pl.ds(w * WG, WG)], p_w)
        pltpu.async_copy(h_hbm.at[src_w.at[0]], rows_a, sem_a)

        def pair(k, _):
            ga = 2 * k
            pltpu.async_copy(h_hbm.at[src_w.at[ga + 1]], rows_b, sem_b)
            pltpu.make_async_copy(h_hbm.at[src_w.at[ga]], rows_a, sem_a).wait()
            scale_scatter(ga, rows_a)

            @pl.when(k + 1 < WG // 2)
            def _():
                pltpu.async_copy(h_hbm.at[src_w.at[ga + 2]], rows_a, sem_a)
            pltpu.make_async_copy(h_hbm.at[src_w.at[ga + 1]], rows_b,
                                  sem_b).wait()
            scale_scatter(ga + 1, rows_b)
            return 0

        lax.fori_loop(0, WG // 2, pair, 0)
        return 0

    lax.fori_loop(0, NWIN, window, 0)
    pltpu.sync_copy(psum_v, psum_hbm.at[wid])
    plsc.subcore_barrier()
    pltpu.sync_copy(acc_sh.at[pl.ds(s * ROWS_PER_TILE, ROWS_PER_TILE)],
                    acc_hbm.at[c, pl.ds(s * ROWS_PER_TILE, ROWS_PER_TILE)])


def _sc_scatter(src_t, dst_t, p_t, h):
    fn = pl.kernel(
        _scatter_body,
        out_type=(
            jax.ShapeDtypeStruct((NC, N2, D_HID), _f32),
            jax.ShapeDtypeStruct((NW, N2), _f32),
        ),
        mesh=_sc_mesh(),
        scratch_types=[
            pltpu.VMEM((WG, G), _i32),
            pltpu.VMEM((WG, G), _i32),
            pltpu.VMEM((WG, G), _f32),
            pltpu.VMEM((G, D_HID), _f32),
            pltpu.VMEM((G, D_HID), _f32),
            pltpu.VMEM((G, D_HID), _f32),
            pltpu.VMEM((N2,), _f32),
            pltpu.VMEM_SHARED((N2, D_HID), _f32),
            pltpu.SemaphoreType.DMA,
            pltpu.SemaphoreType.DMA,
        ],
        compiler_params=_SC_PARAMS,
    )
    return fn(src_t, dst_t, p_t, h)


def _ae_body(ea_ref, vem_ref, ae1_ref, ae2_ref):
    ea = ea_ref[...].astype(jnp.bfloat16).astype(_f32)
    ae1_ref[...] = (ea * vem_ref[0:1, 0:D_EDGE]).sum(-1)
    ae2_ref[...] = (ea * vem_ref[1:2, 0:D_EDGE]).sum(-1)


def _tc_ae(edge_attr_pad, vem):
    grid = E2 // BE
    return pl.pallas_call(
        _ae_body,
        grid=(grid,),
        in_specs=[
            pl.BlockSpec((BE, D_EDGE), lambda i: (i, 0)),
            pl.BlockSpec((8, 128), lambda i: (0, 0)),
        ],
        out_specs=[
            pl.BlockSpec((BE,), lambda i: (i,)),
            pl.BlockSpec((BE,), lambda i: (i,)),
        ],
        out_shape=[
            jax.ShapeDtypeStruct((E2,), _f32),
            jax.ShapeDtypeStruct((E2,), _f32),
        ],
    )(edge_attr_pad, vem)


def _exp_body(s_ref, mb_ref, p_ref):
    s = s_ref[...]
    al = jnp.where(s >= 0.0, s, 0.2 * s)
    p_ref[...] = jnp.exp(al - mb_ref[0, 0:1])


def _tc_exp(s_flat, mb):
    grid = E2 // BE
    return pl.pallas_call(
        _exp_body,
        grid=(grid,),
        in_specs=[
            pl.BlockSpec((BE,), lambda i: (i,)),
            pl.BlockSpec((8, 128), lambda i: (0, 0)),
        ],
        out_specs=pl.BlockSpec((BE,), lambda i: (i,)),
        out_shape=jax.ShapeDtypeStruct((E2,), _f32),
    )(s_flat, mb)


def _n1_body(x_ref, w1t_ref, attm_ref, h1_ref, asrc_ref, adst_ref):
    h = jnp.dot(x_ref[...], w1t_ref[...], preferred_element_type=_f32)
    h1_ref[...] = h
    asrc_ref[...] = (h * attm_ref[0:1, :]).sum(-1)
    adst_ref[...] = (h * attm_ref[1:2, :]).sum(-1)


def _tc_n1(x2, w1t, attm):
    grid = N2 // BN
    vec = jax.ShapeDtypeStruct((N2,), _f32)
    vspec = pl.BlockSpec((BN,), lambda i: (i,))
    return pl.pallas_call(
        _n1_body,
        grid=(grid,),
        in_specs=[
            pl.BlockSpec((BN, D_FEAT), lambda i: (i, 0)),
            pl.BlockSpec((D_FEAT, D_HID), lambda i: (0, 0)),
            pl.BlockSpec((8, 128), lambda i: (0, 0)),
        ],
        out_specs=[pl.BlockSpec((BN, D_HID), lambda i: (i, 0)), vspec, vspec],
        out_shape=[jax.ShapeDtypeStruct((N2, D_HID), _f32), vec, vec],
    )(x2, w1t, attm)


def _n2_body(h1_ref, asrc_ref, adst_ref, mb_ref, acc_ref, stat_ref, psum_ref,
             bw_ref, w2t_ref, attm_ref, h2_ref, asrc2_ref, adst2_ref,
             degc_ref):
    m = mb_ref[0, 0:1]
    st = stat_ref[...].sum(0)
    degc = jnp.maximum(st[:, 1], 1.0)
    degc_ref[...] = degc
    aeL1 = st[:, 0] / degc
    s = asrc_ref[...] + adst_ref[...] + aeL1
    al = jnp.where(s >= 0.0, s, 0.2 * s)
    ps = jnp.exp(al - m)
    num = acc_ref[0] + acc_ref[1] + ps[:, None] * h1_ref[...]
    ssum = psum_ref[...].sum(0) + ps
    o1 = num / (ssum + 1e-16)[:, None] + bw_ref[0:1, :]
    h1r = jnp.maximum(o1, 0.0)
    h2 = jnp.dot(h1r, w2t_ref[...], preferred_element_type=_f32)
    h2_ref[...] = h2
    asrc2_ref[...] = (h2 * attm_ref[0:1, :]).sum(-1)
    adst2_ref[...] = (h2 * attm_ref[1:2, :]).sum(-1)


def _tc_n2(h1, asrc1, adst1, mb1, acc1, stat1, psum1, bw1, w2t, attm2):
    grid = N2 // BN
    vec = jax.ShapeDtypeStruct((N2,), _f32)
    vspec = pl.BlockSpec((BN,), lambda i: (i,))
    return pl.pallas_call(
        _n2_body,
        grid=(grid,),
        in_specs=[
            pl.BlockSpec((BN, D_HID), lambda i: (i, 0)),
            vspec, vspec,
            pl.BlockSpec((8, 128), lambda i: (0, 0)),
            pl.BlockSpec((NC, BN, D_HID), lambda i: (0, i, 0)),
            pl.BlockSpec((NW, BN, 2), lambda i: (0, i, 0)),
            pl.BlockSpec((NW, BN), lambda i: (0, i)),
            pl.BlockSpec((8, 128), lambda i: (0, 0)),
            pl.BlockSpec((D_HID, D_HID), lambda i: (0, 0)),
            pl.BlockSpec((8, 128), lambda i: (0, 0)),
        ],
        out_specs=[
            pl.BlockSpec((BN, D_HID), lambda i: (i, 0)),
            vspec, vspec, vspec,
        ],
        out_shape=[jax.ShapeDtypeStruct((N2, D_HID), _f32), vec, vec, vec],
    )(h1, asrc1, adst1, mb1, acc1, stat1, psum1, bw1, w2t, attm2)


def _n3_body(h2_ref, asrc_ref, adst_ref, degc_ref, mb_ref, acc_ref, stat_ref,
             psum_ref, bw_ref, lin_ref, y_ref):
    m = mb_ref[0, 0:1]
    aeL2 = stat_ref[...].sum(0) / degc_ref[...]
    s = asrc_ref[...] + adst_ref[...] + aeL2
    al = jnp.where(s >= 0.0, s, 0.2 * s)
    ps = jnp.exp(al - m)
    num = acc_ref[0] + acc_ref[1] + ps[:, None] * h2_ref[...]
    ssum = psum_ref[...].sum(0) + ps
    o2 = num / (ssum + 1e-16)[:, None] + bw_ref[0:1, :]
    y = (o2 * lin_ref[0:1, :]).sum(-1) + lin_ref[1, 0:1]
    y_ref[...] = jnp.maximum(y, 0.0)


def _tc_n3(h2, asrc2, adst2, degc, mb2, acc2, stat2, psum2, bw2, linm):
    grid = N2 // BN
    vspec = pl.BlockSpec((BN,), lambda i: (i,))
    return pl.pallas_call(
        _n3_body,
        grid=(grid,),
        in_specs=[
            pl.BlockSpec((BN, D_HID), lambda i: (i, 0)),
            vspec, vspec, vspec,
            pl.BlockSpec((8, 128), lambda i: (0, 0)),
            pl.BlockSpec((NC, BN, D_HID), lambda i: (0, i, 0)),
            pl.BlockSpec((NW, BN), lambda i: (0, i)),
            pl.BlockSpec((NW, BN), lambda i: (0, i)),
            pl.BlockSpec((8, 128), lambda i: (0, 0)),
            pl.BlockSpec((8, 128), lambda i: (0, 0)),
        ],
        out_specs=vspec,
        out_shape=jax.ShapeDtypeStruct((N2,), _f32),
    )(h2, asrc2, adst2, degc, mb2, acc2, stat2, psum2, bw2, linm)


def _pad_rows8(v):
    out = jnp.zeros((8, 128), _f32)
    if v.ndim == 1:
        return out.at[0, :v.shape[0]].set(v)
    return out.at[:v.shape[0], :v.shape[1]].set(v)


def _tile_edges(v, pad_val):
    v = v.reshape(NW, N_EDGES // NW)
    pad = jnp.broadcast_to(pad_val, (NW, EPT - N_EDGES // NW)).astype(v.dtype)
    return jnp.concatenate([v, pad], axis=1).reshape(NW, NG, G)


def _lrelu_scalar(x):
    return jnp.where(x >= 0.0, x, 0.2 * x)


@jax.jit
def kernel(x, edge_index, edge_attr, W1, att_src1, att_dst1, We1, att_e1, b1,
           W2, att_src2, att_dst2, We2, att_e2, b2, linW, linb):
    src = edge_index[0].astype(_i32)
    dst = edge_index[1].astype(_i32)

    ve1 = We1.astype(jnp.bfloat16).astype(_f32).T @ att_e1
    ve2 = We2.astype(jnp.bfloat16).astype(_f32).T @ att_e2
    vem = _pad_rows8(jnp.stack([ve1, ve2]))
    attm1 = _pad_rows8(jnp.stack([att_src1, att_dst1]))
    attm2 = _pad_rows8(jnp.stack([att_src2, att_dst2]))
    bw1 = _pad_rows8(b1)
    bw2 = _pad_rows8(b2)
    linm = _pad_rows8(linW[0]).at[1, 0].set(linb[0])
    x2 = jnp.zeros((N2, D_FEAT), _f32).at[:N_NODES].set(x)

    src_t = _tile_edges(src, 0)
    sent = N_NODES + (jnp.arange(EPT - N_EDGES // NW, dtype=_i32) % 16)
    dst_t = _tile_edges(dst, sent)

    ea_pad = jnp.zeros((E2, D_EDGE), _f32).at[:N_EDGES].set(edge_attr)
    ae1, ae2 = _tc_ae(ea_pad, vem)
    ae1 = ae1[:N_EDGES]
    ae2 = ae2[:N_EDGES]
    ae1_t = _tile_edges(ae1, jnp.float32(-1e30))

    h1, asrc1, adst1 = _tc_n1(x2, W1.T, attm1)

    m1 = _lrelu_scalar(
        jnp.max(asrc1[:N_NODES]) + jnp.max(adst1[:N_NODES])
        + jnp.maximum(jnp.max(ae1), 0.0))
    mb1 = jnp.full((8, 128), m1, _f32)

    s1_t, stat1 = _sc_scores(src_t, dst_t, ae1_t, asrc1, adst1, 2)
    stat1 = stat1.reshape(NW, N2, 2)
    p1_t = _tc_exp(s1_t.reshape(E2), mb1).reshape(NW, NG, G)
    acc1, psum1 = _sc_scatter(src_t, dst_t, p1_t, h1)

    h2, asrc2, adst2, degc = _tc_n2(h1, asrc1, adst1, mb1, acc1, stat1,
                                    psum1, bw1, W2.T, attm2)

    m2 = _lrelu_scalar(
        jnp.max(asrc2[:N_NODES]) + jnp.max(adst2[:N_NODES])
        + jnp.maximum(jnp.max(ae2), 0.0))
    mb2 = jnp.full((8, 128), m2, _f32)

    ae2s_t = _tile_edges(ae2, jnp.float32(-1e30))
    s2_t, stat2 = _sc_scores(src_t, dst_t, ae2s_t, asrc2, adst2, 1)
    stat2 = stat2.reshape(NW, N2)
    p2_t = _tc_exp(s2_t.reshape(E2), mb2).reshape(NW, NG, G)
    acc2, psum2 = _sc_scatter(src_t, dst_t, p2_t, h2)

    y = _tc_n3(h2, asrc2, adst2, degc, mb2, acc2, stat2, psum2, bw2, linm)
    return y[:N_NODES].reshape(N_NODES, 1)


# speedup vs baseline: 12.2045x; 1.0196x over previous
import functools

import jax
import jax.numpy as jnp
from jax import lax
from jax.experimental import pallas as pl
from jax.experimental.pallas import tpu as pltpu
from jax.experimental.pallas import tpu_sc as plsc

N_NODES = 10000
N_EDGES = 320000
D_FEAT = 128
D_HID = 128
D_EDGE = 16

NC = 2
NS = 16
NW = NC * NS
G = 64
EPT = 10240
NG = EPT // G
WG = 16
NWIN = NG // WG
N2 = 10240
ROWS_PER_TILE = N2 // NS
BN = 512
BE = 4096
E2 = 327680

_f32 = jnp.float32
_i32 = jnp.int32


def _sc_mesh():
    return plsc.VectorSubcoreMesh(core_axis_name="c", subcore_axis_name="s",
                                  num_cores=NC, num_subcores=NS)


_SC_PARAMS = pltpu.CompilerParams(needs_layout_passes=False)


def _splat(vec, lane):
    return jnp.take_along_axis(vec, jnp.full((16,), lane, _i32), axis=0,
                               mode="promise_in_bounds")


def _score_body(nstat,
                src_hbm, dst_hbm, ae_hbm, asrc_hbm, adst_hbm,
                s_hbm, stat_hbm,
                src_w, dst_w, ae_w, asrc_v, adst_v, s_w, stat_v):
    c = lax.axis_index("c")
    s = lax.axis_index("s")
    wid = s * NC + c

    pltpu.sync_copy(asrc_hbm, asrc_v)
    pltpu.sync_copy(adst_hbm, adst_v)

    zero16 = jnp.zeros((16,), _f32)

    def zstat(i, _):
        stat_v[pl.ds(i * 16, 16)] = zero16
        return 0
    lax.fori_loop(0, (N2 * nstat) // 16, zstat, 0)

    iota = lax.iota(_i32, 16)
    ones = jnp.ones((16,), _f32)
    stat_mask = iota < nstat

    def group(gg, _):
        for i in range(G // 16):
            sv = src_w[gg, pl.ds(16 * i, 16)]
            dv = dst_w[gg, pl.ds(16 * i, 16)]
            aev = ae_w[gg, pl.ds(16 * i, 16)]
            s_w[gg, pl.ds(16 * i, 16)] = (
                plsc.load_gather(asrc_v, [sv])
                + plsc.load_gather(adst_v, [dv])
                + aev)
            for l in range(16):
                idxs = _splat(dv, l) * nstat + iota
                if nstat == 2:
                    val = jnp.where(iota == 0, _splat(aev, l), ones)
                else:
                    val = _splat(aev, l)
                plsc.addupdate_scatter(stat_v, [idxs], val, mask=stat_mask)
        return 0

    def window(w, _):
        pltpu.sync_copy(src_hbm.at[wid, pl.ds(w * WG, WG)], src_w)
        pltpu.sync_copy(dst_hbm.at[wid, pl.ds(w * WG, WG)], dst_w)
        pltpu.sync_copy(ae_hbm.at[wid, pl.ds(w * WG, WG)], ae_w)
        lax.fori_loop(0, WG, group, 0)
        pltpu.sync_copy(s_w, s_hbm.at[wid, pl.ds(w * WG, WG)])
        return 0

    lax.fori_loop(0, NWIN, window, 0)
    pltpu.sync_copy(stat_v, stat_hbm.at[wid])


def _sc_scores(src_t, dst_t, ae_t, asrc, adst, nstat):
    fn = pl.kernel(
        functools.partial(_score_body, nstat),
        out_type=(
            jax.ShapeDtypeStruct((NW, NG, G), _f32),
            jax.ShapeDtypeStruct((NW, N2 * nstat), _f32),
        ),
        mesh=_sc_mesh(),
        scratch_types=[
            pltpu.VMEM((WG, G), _i32),
            pltpu.VMEM((WG, G), _i32),
            pltpu.VMEM((WG, G), _f32),
            pltpu.VMEM((N2,), _f32),
            pltpu.VMEM((N2,), _f32),
            pltpu.VMEM((WG, G), _f32),
            pltpu.VMEM((N2 * nstat,), _f32),
        ],
        compiler_params=_SC_PARAMS,
    )
    return fn(src_t, dst_t, ae_t, asrc, adst)


def _scatter_body(src_hbm, dst_hbm, p_hbm, h_hbm, acc_hbm, psum_hbm,
                  src_w, dst_w, p_w, rows_a, rows_b, ext_a, ext_b, psum_v,
                  acc_sh, sem_a, sem_b, sem_c):
    c = lax.axis_index("c")
    s = lax.axis_index("s")
    wid = s * NC + c

    zero16 = jnp.zeros((16,), _f32)

    def zps(i, _):
        psum_v[pl.ds(i * 16, 16)] = zero16
        return 0
    lax.fori_loop(0, N2 // 16, zps, 0)

    for e in range(G):
        for j in range(D_HID // 16):
            ext_a[e, pl.ds(16 * j, 16)] = zero16
    for k in range(ROWS_PER_TILE // G):
        pltpu.sync_copy(ext_a, acc_sh.at[pl.ds(s * ROWS_PER_TILE + k * G, G)])
    plsc.subcore_barrier()

    iota = lax.iota(_i32, 16)
    lane0 = iota == 0

    def scale(gg, rows, ext):
        for i in range(G // 16):
            pvec = p_w[gg, pl.ds(16 * i, 16)]
            dvec = dst_w[gg, pl.ds(16 * i, 16)]
            for l in range(16):
                e = i * 16 + l
                p16 = _splat(pvec, l)
                for j in range(D_HID // 16):
                    ext[e, pl.ds(16 * j, 16)] = (
                        rows[e, pl.ds(16 * j, 16)] * p16)
                plsc.addupdate_scatter(psum_v, [_splat(dvec, l) + iota], p16,
                                       mask=lane0)

    def drain_scatters(gg):
        pltpu.make_async_copy(ext_a, acc_sh.at[dst_w.at[gg]], sem_c).wait()
        pltpu.make_async_copy(ext_b, acc_sh.at[dst_w.at[gg]], sem_c).wait()

    def window(w, _):
        pltpu.sync_copy(src_hbm.at[wid, pl.ds(w * WG, WG)], src_w)
        pltpu.sync_copy(dst_hbm.at[wid, pl.ds(w * WG, WG)], dst_w)
        pltpu.sync_copy(p_hbm.at[wid, pl.ds(w * WG, WG)], p_w)
        pltpu.async_copy(h_hbm.at[src_w.at[0]], rows_a, sem_a)

        def pair(k, _):
            ga = 2 * k

            @pl.when(k > 0)
            def _():
                drain_scatters(ga)
            pltpu.async_copy(h_hbm.at[src_w.at[ga + 1]], rows_b, sem_b)
            pltpu.make_async_copy(h_hbm.at[src_w.at[ga]], rows_a, sem_a).wait()
            scale(ga, rows_a, ext_a)
            pltpu.async_copy(ext_a, acc_sh.at[dst_w.at[ga]], sem_c, add=True)

            @pl.when(k + 1 < WG // 2)
            def _():
                pltpu.async_copy(h_hbm.at[src_w.at[ga + 2]], rows_a, sem_a)
            pltpu.make_async_copy(h_hbm.at[src_w.at[ga + 1]], rows_b,
                                  sem_b).wait()
            scale(ga + 1, rows_b, ext_b)
            pltpu.async_copy(ext_b, acc_sh.at[dst_w.at[ga + 1]], sem_c,
                             add=True)
            return 0

        lax.fori_loop(0, WG // 2, pair, 0)
        drain_scatters(0)
        return 0

    lax.fori_loop(0, NWIN, window, 0)
    pltpu.sync_copy(psum_v, psum_hbm.at[wid])
    plsc.subcore_barrier()
    pltpu.sync_copy(acc_sh.at[pl.ds(s * ROWS_PER_TILE, ROWS_PER_TILE)],
                    acc_hbm.at[c, pl.ds(s * ROWS_PER_TILE, ROWS_PER_TILE)])


def _sc_scatter(src_t, dst_t, p_t, h):
    fn = pl.kernel(
        _scatter_body,
        out_type=(
            jax.ShapeDtypeStruct((NC, N2, D_HID), _f32),
            jax.ShapeDtypeStruct((NW, N2), _f32),
        ),
        mesh=_sc_mesh(),
        scratch_types=[
            pltpu.VMEM((WG, G), _i32),
            pltpu.VMEM((WG, G), _i32),
            pltpu.VMEM((WG, G), _f32),
            pltpu.VMEM((G, D_HID), _f32),
            pltpu.VMEM((G, D_HID), _f32),
            pltpu.VMEM((G, D_HID), _f32),
            pltpu.VMEM((G, D_HID), _f32),
            pltpu.VMEM((N2,), _f32),
            pltpu.VMEM_SHARED((N2, D_HID), _f32),
            pltpu.SemaphoreType.DMA,
            pltpu.SemaphoreType.DMA,
            pltpu.SemaphoreType.DMA,
        ],
        compiler_params=_SC_PARAMS,
    )
    return fn(src_t, dst_t, p_t, h)


def _ae_body(ea_ref, vem_ref, ae1_ref, ae2_ref):
    ea = ea_ref[...].astype(jnp.bfloat16).astype(_f32)
    ae1_ref[...] = (ea * vem_ref[0:1, 0:D_EDGE]).sum(-1)
    ae2_ref[...] = (ea * vem_ref[1:2, 0:D_EDGE]).sum(-1)


def _tc_ae(edge_attr_pad, vem):
    grid = E2 // BE
    return pl.pallas_call(
        _ae_body,
        grid=(grid,),
        in_specs=[
            pl.BlockSpec((BE, D_EDGE), lambda i: (i, 0)),
            pl.BlockSpec((8, 128), lambda i: (0, 0)),
        ],
        out_specs=[
            pl.BlockSpec((BE,), lambda i: (i,)),
            pl.BlockSpec((BE,), lambda i: (i,)),
        ],
        out_shape=[
            jax.ShapeDtypeStruct((E2,), _f32),
            jax.ShapeDtypeStruct((E2,), _f32),
        ],
    )(edge_attr_pad, vem)


def _exp_body(s_ref, mb_ref, p_ref):
    s = s_ref[...]
    al = jnp.where(s >= 0.0, s, 0.2 * s)
    p_ref[...] = jnp.exp(al - mb_ref[0, 0:1])


def _tc_exp(s_flat, mb):
    grid = E2 // BE
    return pl.pallas_call(
        _exp_body,
        grid=(grid,),
        in_specs=[
            pl.BlockSpec((BE,), lambda i: (i,)),
            pl.BlockSpec((8, 128), lambda i: (0, 0)),
        ],
        out_specs=pl.BlockSpec((BE,), lambda i: (i,)),
        out_shape=jax.ShapeDtypeStruct((E2,), _f32),
    )(s_flat, mb)


def _n1_body(x_ref, w1t_ref, attm_ref, h1_ref, asrc_ref, adst_ref):
    h = jnp.dot(x_ref[...], w1t_ref[...], preferred_element_type=_f32)
    h1_ref[...] = h
    asrc_ref[...] = (h * attm_ref[0:1, :]).sum(-1)
    adst_ref[...] = (h * attm_ref[1:2, :]).sum(-1)


def _tc_n1(x2, w1t, attm):
    grid = N2 // BN
    vec = jax.ShapeDtypeStruct((N2,), _f32)
    vspec = pl.BlockSpec((BN,), lambda i: (i,))
    return pl.pallas_call(
        _n1_body,
        grid=(grid,),
        in_specs=[
            pl.BlockSpec((BN, D_FEAT), lambda i: (i, 0)),
            pl.BlockSpec((D_FEAT, D_HID), lambda i: (0, 0)),
            pl.BlockSpec((8, 128), lambda i: (0, 0)),
        ],
        out_specs=[pl.BlockSpec((BN, D_HID), lambda i: (i, 0)), vspec, vspec],
        out_shape=[jax.ShapeDtypeStruct((N2, D_HID), _f32), vec, vec],
    )(x2, w1t, attm)


def _n2_body(h1_ref, asrc_ref, adst_ref, mb_ref, acc_ref, stat_ref, psum_ref,
             bw_ref, w2t_ref, attm_ref, h2_ref, asrc2_ref, adst2_ref,
             degc_ref):
    m = mb_ref[0, 0:1]
    st = stat_ref[...].sum(0)
    degc = jnp.maximum(st[:, 1], 1.0)
    degc_ref[...] = degc
    aeL1 = st[:, 0] / degc
    s = asrc_ref[...] + adst_ref[...] + aeL1
    al = jnp.where(s >= 0.0, s, 0.2 * s)
    ps = jnp.exp(al - m)
    num = acc_ref[0] + acc_ref[1] + ps[:, None] * h1_ref[...]
    ssum = psum_ref[...].sum(0) + ps
    o1 = num / (ssum + 1e-16)[:, None] + bw_ref[0:1, :]
    h1r = jnp.maximum(o1, 0.0)
    h2 = jnp.dot(h1r, w2t_ref[...], preferred_element_type=_f32)
    h2_ref[...] = h2
    asrc2_ref[...] = (h2 * attm_ref[0:1, :]).sum(-1)
    adst2_ref[...] = (h2 * attm_ref[1:2, :]).sum(-1)


def _tc_n2(h1, asrc1, adst1, mb1, acc1, stat1, psum1, bw1, w2t, attm2):
    grid = N2 // BN
    vec = jax.ShapeDtypeStruct((N2,), _f32)
    vspec = pl.BlockSpec((BN,), lambda i: (i,))
    return pl.pallas_call(
        _n2_body,
        grid=(grid,),
        in_specs=[
            pl.BlockSpec((BN, D_HID), lambda i: (i, 0)),
            vspec, vspec,
            pl.BlockSpec((8, 128), lambda i: (0, 0)),
            pl.BlockSpec((NC, BN, D_HID), lambda i: (0, i, 0)),
            pl.BlockSpec((NW, BN, 2), lambda i: (0, i, 0)),
            pl.BlockSpec((NW, BN), lambda i: (0, i)),
            pl.BlockSpec((8, 128), lambda i: (0, 0)),
            pl.BlockSpec((D_HID, D_HID), lambda i: (0, 0)),
            pl.BlockSpec((8, 128), lambda i: (0, 0)),
        ],
        out_specs=[
            pl.BlockSpec((BN, D_HID), lambda i: (i, 0)),
            vspec, vspec, vspec,
        ],
        out_shape=[jax.ShapeDtypeStruct((N2, D_HID), _f32), vec, vec, vec],
    )(h1, asrc1, adst1, mb1, acc1, stat1, psum1, bw1, w2t, attm2)


def _n3_body(h2_ref, asrc_ref, adst_ref, degc_ref, mb_ref, acc_ref, stat_ref,
             psum_ref, bw_ref, lin_ref, y_ref):
    m = mb_ref[0, 0:1]
    aeL2 = stat_ref[...].sum(0) / degc_ref[...]
    s = asrc_ref[...] + adst_ref[...] + aeL2
    al = jnp.where(s >= 0.0, s, 0.2 * s)
    ps = jnp.exp(al - m)
    num = acc_ref[0] + acc_ref[1] + ps[:, None] * h2_ref[...]
    ssum = psum_ref[...].sum(0) + ps
    o2 = num / (ssum + 1e-16)[:, None] + bw_ref[0:1, :]
    y = (o2 * lin_ref[0:1, :]).sum(-1) + lin_ref[1, 0:1]
    y_ref[...] = jnp.maximum(y, 0.0)


def _tc_n3(h2, asrc2, adst2, degc, mb2, acc2, stat2, psum2, bw2, linm):
    grid = N2 // BN
    vspec = pl.BlockSpec((BN,), lambda i: (i,))
    return pl.pallas_call(
        _n3_body,
        grid=(grid,),
        in_specs=[
            pl.BlockSpec((BN, D_HID), lambda i: (i, 0)),
            vspec, vspec, vspec,
            pl.BlockSpec((8, 128), lambda i: (0, 0)),
            pl.BlockSpec((NC, BN, D_HID), lambda i: (0, i, 0)),
            pl.BlockSpec((NW, BN), lambda i: (0, i)),
            pl.BlockSpec((NW, BN), lambda i: (0, i)),
            pl.BlockSpec((8, 128), lambda i: (0, 0)),
            pl.BlockSpec((8, 128), lambda i: (0, 0)),
        ],
        out_specs=vspec,
        out_shape=jax.ShapeDtypeStruct((N2,), _f32),
    )(h2, asrc2, adst2, degc, mb2, acc2, stat2, psum2, bw2, linm)


def _pad_rows8(v):
    out = jnp.zeros((8, 128), _f32)
    if v.ndim == 1:
        return out.at[0, :v.shape[0]].set(v)
    return out.at[:v.shape[0], :v.shape[1]].set(v)


def _tile_edges(v, pad_val):
    v = v.reshape(NW, N_EDGES // NW)
    pad = jnp.broadcast_to(pad_val, (NW, EPT - N_EDGES // NW)).astype(v.dtype)
    return jnp.concatenate([v, pad], axis=1).reshape(NW, NG, G)


def _lrelu_scalar(x):
    return jnp.where(x >= 0.0, x, 0.2 * x)


@jax.jit
def kernel(x, edge_index, edge_attr, W1, att_src1, att_dst1, We1, att_e1, b1,
           W2, att_src2, att_dst2, We2, att_e2, b2, linW, linb):
    src = edge_index[0].astype(_i32)
    dst = edge_index[1].astype(_i32)

    ve1 = We1.astype(jnp.bfloat16).astype(_f32).T @ att_e1
    ve2 = We2.astype(jnp.bfloat16).astype(_f32).T @ att_e2
    vem = _pad_rows8(jnp.stack([ve1, ve2]))
    attm1 = _pad_rows8(jnp.stack([att_src1, att_dst1]))
    attm2 = _pad_rows8(jnp.stack([att_src2, att_dst2]))
    bw1 = _pad_rows8(b1)
    bw2 = _pad_rows8(b2)
    linm = _pad_rows8(linW[0]).at[1, 0].set(linb[0])
    x2 = jnp.zeros((N2, D_FEAT), _f32).at[:N_NODES].set(x)

    src_t = _tile_edges(src, 0)
    sent = N_NODES + (jnp.arange(EPT - N_EDGES // NW, dtype=_i32) % 16)
    dst_t = _tile_edges(dst, sent)

    ea_pad = jnp.zeros((E2, D_EDGE), _f32).at[:N_EDGES].set(edge_attr)
    ae1, ae2 = _tc_ae(ea_pad, vem)
    ae1 = ae1[:N_EDGES]
    ae2 = ae2[:N_EDGES]
    ae1_t = _tile_edges(ae1, jnp.float32(-1e30))

    h1, asrc1, adst1 = _tc_n1(x2, W1.T, attm1)

    m1 = _lrelu_scalar(
        jnp.max(asrc1[:N_NODES]) + jnp.max(adst1[:N_NODES])
        + jnp.maximum(jnp.max(ae1), 0.0))
    mb1 = jnp.full((8, 128), m1, _f32)

    s1_t, stat1 = _sc_scores(src_t, dst_t, ae1_t, asrc1, adst1, 2)
    stat1 = stat1.reshape(NW, N2, 2)
    p1_t = _tc_exp(s1_t.reshape(E2), mb1).reshape(NW, NG, G)
    acc1, psum1 = _sc_scatter(src_t, dst_t, p1_t, h1)

    h2, asrc2, adst2, degc = _tc_n2(h1, asrc1, adst1, mb1, acc1, stat1,
                                    psum1, bw1, W2.T, attm2)

    m2 = _lrelu_scalar(
        jnp.max(asrc2[:N_NODES]) + jnp.max(adst2[:N_NODES])
        + jnp.maximum(jnp.max(ae2), 0.0))
    mb2 = jnp.full((8, 128), m2, _f32)

    ae2s_t = _tile_edges(ae2, jnp.float32(-1e30))
    s2_t, stat2 = _sc_scores(src_t, dst_t, ae2s_t, asrc2, adst2, 1)
    stat2 = stat2.reshape(NW, N2)
    p2_t = _tc_exp(s2_t.reshape(E2), mb2).reshape(NW, NG, G)
    acc2, psum2 = _sc_scatter(src_t, dst_t, p2_t, h2)

    y = _tc_n3(h2, asrc2, adst2, degc, mb2, acc2, stat2, psum2, bw2, linm)
    return y[:N_NODES].reshape(N_NODES, 1)


# speedup vs baseline: 13.2388x; 1.0847x over previous
import functools

import jax
import jax.numpy as jnp
from jax import lax
from jax.experimental import pallas as pl
from jax.experimental.pallas import tpu as pltpu
from jax.experimental.pallas import tpu_sc as plsc

N_NODES = 10000
N_EDGES = 320000
D_FEAT = 128
D_HID = 128
D_EDGE = 16

NC = 2
NS = 16
NW = NC * NS
G = 64
EPT = 10240
NG = EPT // G
WG = 16
NWIN = NG // WG
N2 = 10240
ROWS_PER_TILE = N2 // NS
BN = 512
BE = 4096
E2 = 327680

_f32 = jnp.float32
_i32 = jnp.int32


def _sc_mesh():
    return plsc.VectorSubcoreMesh(core_axis_name="c", subcore_axis_name="s",
                                  num_cores=NC, num_subcores=NS)


_SC_PARAMS = pltpu.CompilerParams(needs_layout_passes=False)


def _splat(vec, lane):
    return jnp.take_along_axis(vec, jnp.full((16,), lane, _i32), axis=0,
                               mode="promise_in_bounds")


def _score_body(nstat,
                src_hbm, dst_hbm, ae_hbm, asrc_hbm, adst_hbm,
                s_hbm, stat_hbm,
                src_w, dst_w, ae_w, asrc_v, adst_v, s_w, stat_v):
    c = lax.axis_index("c")
    s = lax.axis_index("s")
    wid = s * NC + c

    pltpu.sync_copy(asrc_hbm, asrc_v)
    pltpu.sync_copy(adst_hbm, adst_v)

    zero16 = jnp.zeros((16,), _f32)

    def zstat(i, _):
        stat_v[pl.ds(i * 16, 16)] = zero16
        return 0
    lax.fori_loop(0, (N2 * nstat) // 16, zstat, 0)

    iota = lax.iota(_i32, 16)
    ones = jnp.ones((16,), _f32)
    stat_mask = iota < nstat

    def group(gg, _):
        for i in range(G // 16):
            sv = src_w[gg, pl.ds(16 * i, 16)]
            dv = dst_w[gg, pl.ds(16 * i, 16)]
            aev = ae_w[gg, pl.ds(16 * i, 16)]
            s_w[gg, pl.ds(16 * i, 16)] = (
                plsc.load_gather(asrc_v, [sv])
                + plsc.load_gather(adst_v, [dv])
                + aev)
            for l in range(16):
                idxs = _splat(dv, l) * nstat + iota
                if nstat == 2:
                    val = jnp.where(iota == 0, _splat(aev, l), ones)
                else:
                    val = _splat(aev, l)
                plsc.addupdate_scatter(stat_v, [idxs], val, mask=stat_mask)
        return 0

    def window(w, _):
        pltpu.sync_copy(src_hbm.at[wid, pl.ds(w * WG, WG)], src_w)
        pltpu.sync_copy(dst_hbm.at[wid, pl.ds(w * WG, WG)], dst_w)
        pltpu.sync_copy(ae_hbm.at[wid, pl.ds(w * WG, WG)], ae_w)
        lax.fori_loop(0, WG, group, 0)
        pltpu.sync_copy(s_w, s_hbm.at[wid, pl.ds(w * WG, WG)])
        return 0

    lax.fori_loop(0, NWIN, window, 0)
    pltpu.sync_copy(stat_v, stat_hbm.at[wid])


def _sc_scores(src_t, dst_t, ae_t, asrc, adst, nstat):
    fn = pl.kernel(
        functools.partial(_score_body, nstat),
        out_type=(
            jax.ShapeDtypeStruct((NW, NG, G), _f32),
            jax.ShapeDtypeStruct((NW, N2 * nstat), _f32),
        ),
        mesh=_sc_mesh(),
        scratch_types=[
            pltpu.VMEM((WG, G), _i32),
            pltpu.VMEM((WG, G), _i32),
            pltpu.VMEM((WG, G), _f32),
            pltpu.VMEM((N2,), _f32),
            pltpu.VMEM((N2,), _f32),
            pltpu.VMEM((WG, G), _f32),
            pltpu.VMEM((N2 * nstat,), _f32),
        ],
        compiler_params=_SC_PARAMS,
    )
    return fn(src_t, dst_t, ae_t, asrc, adst)


def _scatter_body(src_hbm, dst_hbm, p_hbm, h_hbm, acc_hbm, psum_hbm,
                  src_w, dst_w, p_w, rows_a, rows_b, ext_a, ext_b, psum_v,
                  acc_sh, sem_a, sem_b, sem_c):
    c = lax.axis_index("c")
    s = lax.axis_index("s")
    wid = s * NC + c

    zero16 = jnp.zeros((16,), _f32)

    def zps(i, _):
        psum_v[pl.ds(i * 16, 16)] = zero16
        return 0
    lax.fori_loop(0, N2 // 16, zps, 0)

    for e in range(G):
        for j in range(D_HID // 16):
            ext_a[e, pl.ds(16 * j, 16)] = zero16
    for k in range(ROWS_PER_TILE // G):
        pltpu.sync_copy(ext_a, acc_sh.at[pl.ds(s * ROWS_PER_TILE + k * G, G)])
    plsc.subcore_barrier()

    iota = lax.iota(_i32, 16)
    lane0 = iota == 0

    def scale(gg, rows, ext):
        def chunk(i, _):
            pvec = p_w[gg, pl.ds(16 * i, 16)]
            dvec = dst_w[gg, pl.ds(16 * i, 16)]
            for l in range(16):
                e = 16 * i + l
                p16 = _splat(pvec, l)
                for j in range(D_HID // 16):
                    ext[e, pl.ds(16 * j, 16)] = (
                        rows[e, pl.ds(16 * j, 16)] * p16)
                plsc.addupdate_scatter(psum_v, [_splat(dvec, l) + iota], p16,
                                       mask=lane0)
            return 0
        lax.fori_loop(0, G // 16, chunk, 0)

    def drain_scatters(gg):
        pltpu.make_async_copy(ext_a, acc_sh.at[dst_w.at[gg]], sem_c).wait()
        pltpu.make_async_copy(ext_b, acc_sh.at[dst_w.at[gg]], sem_c).wait()

    def window(w, _):
        pltpu.sync_copy(src_hbm.at[wid, pl.ds(w * WG, WG)], src_w)
        pltpu.sync_copy(dst_hbm.at[wid, pl.ds(w * WG, WG)], dst_w)
        pltpu.sync_copy(p_hbm.at[wid, pl.ds(w * WG, WG)], p_w)
        pltpu.async_copy(h_hbm.at[src_w.at[0]], rows_a, sem_a)

        def pair(k, _):
            ga = 2 * k

            @pl.when(k > 0)
            def _():
                drain_scatters(ga)
            pltpu.async_copy(h_hbm.at[src_w.at[ga + 1]], rows_b, sem_b)
            pltpu.make_async_copy(h_hbm.at[src_w.at[ga]], rows_a, sem_a).wait()
            scale(ga, rows_a, ext_a)
            pltpu.async_copy(ext_a, acc_sh.at[dst_w.at[ga]], sem_c, add=True)

            @pl.when(k + 1 < WG // 2)
            def _():
                pltpu.async_copy(h_hbm.at[src_w.at[ga + 2]], rows_a, sem_a)
            pltpu.make_async_copy(h_hbm.at[src_w.at[ga + 1]], rows_b,
                                  sem_b).wait()
            scale(ga + 1, rows_b, ext_b)
            pltpu.async_copy(ext_b, acc_sh.at[dst_w.at[ga + 1]], sem_c,
                             add=True)
            return 0

        lax.fori_loop(0, WG // 2, pair, 0)
        drain_scatters(0)
        return 0

    lax.fori_loop(0, NWIN, window, 0)
    pltpu.sync_copy(psum_v, psum_hbm.at[wid])
    plsc.subcore_barrier()
    pltpu.sync_copy(acc_sh.at[pl.ds(s * ROWS_PER_TILE, ROWS_PER_TILE)],
                    acc_hbm.at[c, pl.ds(s * ROWS_PER_TILE, ROWS_PER_TILE)])


def _sc_scatter(src_t, dst_t, p_t, h):
    fn = pl.kernel(
        _scatter_body,
        out_type=(
            jax.ShapeDtypeStruct((NC, N2, D_HID), _f32),
            jax.ShapeDtypeStruct((NW, N2), _f32),
        ),
        mesh=_sc_mesh(),
        scratch_types=[
            pltpu.VMEM((WG, G), _i32),
            pltpu.VMEM((WG, G), _i32),
            pltpu.VMEM((WG, G), _f32),
            pltpu.VMEM((G, D_HID), _f32),
            pltpu.VMEM((G, D_HID), _f32),
            pltpu.VMEM((G, D_HID), _f32),
            pltpu.VMEM((G, D_HID), _f32),
            pltpu.VMEM((N2,), _f32),
            pltpu.VMEM_SHARED((N2, D_HID), _f32),
            pltpu.SemaphoreType.DMA,
            pltpu.SemaphoreType.DMA,
            pltpu.SemaphoreType.DMA,
        ],
        compiler_params=_SC_PARAMS,
    )
    return fn(src_t, dst_t, p_t, h)


def _ae_body(ea_ref, vem_ref, ae1_ref, ae2_ref):
    ea = ea_ref[...].astype(jnp.bfloat16).astype(_f32)
    ae1_ref[...] = (ea * vem_ref[0:1, 0:D_EDGE]).sum(-1)
    ae2_ref[...] = (ea * vem_ref[1:2, 0:D_EDGE]).sum(-1)


def _tc_ae(edge_attr_pad, vem):
    grid = E2 // BE
    return pl.pallas_call(
        _ae_body,
        grid=(grid,),
        in_specs=[
            pl.BlockSpec((BE, D_EDGE), lambda i: (i, 0)),
            pl.BlockSpec((8, 128), lambda i: (0, 0)),
        ],
        out_specs=[
            pl.BlockSpec((BE,), lambda i: (i,)),
            pl.BlockSpec((BE,), lambda i: (i,)),
        ],
        out_shape=[
            jax.ShapeDtypeStruct((E2,), _f32),
            jax.ShapeDtypeStruct((E2,), _f32),
        ],
    )(edge_attr_pad, vem)


def _exp_body(s_ref, mb_ref, p_ref):
    s = s_ref[...]
    al = jnp.where(s >= 0.0, s, 0.2 * s)
    p_ref[...] = jnp.exp(al - mb_ref[0, 0:1])


def _tc_exp(s_flat, mb):
    grid = E2 // BE
    return pl.pallas_call(
        _exp_body,
        grid=(grid,),
        in_specs=[
            pl.BlockSpec((BE,), lambda i: (i,)),
            pl.BlockSpec((8, 128), lambda i: (0, 0)),
        ],
        out_specs=pl.BlockSpec((BE,), lambda i: (i,)),
        out_shape=jax.ShapeDtypeStruct((E2,), _f32),
    )(s_flat, mb)


def _n1_body(x_ref, w1t_ref, attm_ref, h1_ref, asrc_ref, adst_ref):
    h = jnp.dot(x_ref[...], w1t_ref[...], preferred_element_type=_f32)
    h1_ref[...] = h
    asrc_ref[...] = (h * attm_ref[0:1, :]).sum(-1)
    adst_ref[...] = (h * attm_ref[1:2, :]).sum(-1)


def _tc_n1(x2, w1t, attm):
    grid = N2 // BN
    vec = jax.ShapeDtypeStruct((N2,), _f32)
    vspec = pl.BlockSpec((BN,), lambda i: (i,))
    return pl.pallas_call(
        _n1_body,
        grid=(grid,),
        in_specs=[
            pl.BlockSpec((BN, D_FEAT), lambda i: (i, 0)),
            pl.BlockSpec((D_FEAT, D_HID), lambda i: (0, 0)),
            pl.BlockSpec((8, 128), lambda i: (0, 0)),
        ],
        out_specs=[pl.BlockSpec((BN, D_HID), lambda i: (i, 0)), vspec, vspec],
        out_shape=[jax.ShapeDtypeStruct((N2, D_HID), _f32), vec, vec],
    )(x2, w1t, attm)


def _n2_body(h1_ref, asrc_ref, adst_ref, mb_ref, acc_ref, stat_ref, psum_ref,
             bw_ref, w2t_ref, attm_ref, h2_ref, asrc2_ref, adst2_ref,
             degc_ref):
    m = mb_ref[0, 0:1]
    st = stat_ref[...].sum(0)
    degc = jnp.maximum(st[:, 1], 1.0)
    degc_ref[...] = degc
    aeL1 = st[:, 0] / degc
    s = asrc_ref[...] + adst_ref[...] + aeL1
    al = jnp.where(s >= 0.0, s, 0.2 * s)
    ps = jnp.exp(al - m)
    num = acc_ref[0] + acc_ref[1] + ps[:, None] * h1_ref[...]
    ssum = psum_ref[...].sum(0) + ps
    o1 = num / (ssum + 1e-16)[:, None] + bw_ref[0:1, :]
    h1r = jnp.maximum(o1, 0.0)
    h2 = jnp.dot(h1r, w2t_ref[...], preferred_element_type=_f32)
    h2_ref[...] = h2
    asrc2_ref[...] = (h2 * attm_ref[0:1, :]).sum(-1)
    adst2_ref[...] = (h2 * attm_ref[1:2, :]).sum(-1)


def _tc_n2(h1, asrc1, adst1, mb1, acc1, stat1, psum1, bw1, w2t, attm2):
    grid = N2 // BN
    vec = jax.ShapeDtypeStruct((N2,), _f32)
    vspec = pl.BlockSpec((BN,), lambda i: (i,))
    return pl.pallas_call(
        _n2_body,
        grid=(grid,),
        in_specs=[
            pl.BlockSpec((BN, D_HID), lambda i: (i, 0)),
            vspec, vspec,
            pl.BlockSpec((8, 128), lambda i: (0, 0)),
            pl.BlockSpec((NC, BN, D_HID), lambda i: (0, i, 0)),
            pl.BlockSpec((NW, BN, 2), lambda i: (0, i, 0)),
            pl.BlockSpec((NW, BN), lambda i: (0, i)),
            pl.BlockSpec((8, 128), lambda i: (0, 0)),
            pl.BlockSpec((D_HID, D_HID), lambda i: (0, 0)),
            pl.BlockSpec((8, 128), lambda i: (0, 0)),
        ],
        out_specs=[
            pl.BlockSpec((BN, D_HID), lambda i: (i, 0)),
            vspec, vspec, vspec,
        ],
        out_shape=[jax.ShapeDtypeStruct((N2, D_HID), _f32), vec, vec, vec],
    )(h1, asrc1, adst1, mb1, acc1, stat1, psum1, bw1, w2t, attm2)


def _n3_body(h2_ref, asrc_ref, adst_ref, degc_ref, mb_ref, acc_ref, stat_ref,
             psum_ref, bw_ref, lin_ref, y_ref):
    m = mb_ref[0, 0:1]
    aeL2 = stat_ref[...].sum(0) / degc_ref[...]
    s = asrc_ref[...] + adst_ref[...] + aeL2
    al = jnp.where(s >= 0.0, s, 0.2 * s)
    ps = jnp.exp(al - m)
    num = acc_ref[0] + acc_ref[1] + ps[:, None] * h2_ref[...]
    ssum = psum_ref[...].sum(0) + ps
    o2 = num / (ssum + 1e-16)[:, None] + bw_ref[0:1, :]
    y = (o2 * lin_ref[0:1, :]).sum(-1) + lin_ref[1, 0:1]
    y_ref[...] = jnp.maximum(y, 0.0)


def _tc_n3(h2, asrc2, adst2, degc, mb2, acc2, stat2, psum2, bw2, linm):
    grid = N2 // BN
    vspec = pl.BlockSpec((BN,), lambda i: (i,))
    return pl.pallas_call(
        _n3_body,
        grid=(grid,),
        in_specs=[
            pl.BlockSpec((BN, D_HID), lambda i: (i, 0)),
            vspec, vspec, vspec,
            pl.BlockSpec((8, 128), lambda i: (0, 0)),
            pl.BlockSpec((NC, BN, D_HID), lambda i: (0, i, 0)),
            pl.BlockSpec((NW, BN), lambda i: (0, i)),
            pl.BlockSpec((NW, BN), lambda i: (0, i)),
            pl.BlockSpec((8, 128), lambda i: (0, 0)),
            pl.BlockSpec((8, 128), lambda i: (0, 0)),
        ],
        out_specs=vspec,
        out_shape=jax.ShapeDtypeStruct((N2,), _f32),
    )(h2, asrc2, adst2, degc, mb2, acc2, stat2, psum2, bw2, linm)


def _pad_rows8(v):
    out = jnp.zeros((8, 128), _f32)
    if v.ndim == 1:
        return out.at[0, :v.shape[0]].set(v)
    return out.at[:v.shape[0], :v.shape[1]].set(v)


def _tile_edges(v, pad_val):
    v = v.reshape(NW, N_EDGES // NW)
    pad = jnp.broadcast_to(pad_val, (NW, EPT - N_EDGES // NW)).astype(v.dtype)
    return jnp.concatenate([v, pad], axis=1).reshape(NW, NG, G)


def _lrelu_scalar(x):
    return jnp.where(x >= 0.0, x, 0.2 * x)


@jax.jit
def kernel(x, edge_index, edge_attr, W1, att_src1, att_dst1, We1, att_e1, b1,
           W2, att_src2, att_dst2, We2, att_e2, b2, linW, linb):
    src = edge_index[0].astype(_i32)
    dst = edge_index[1].astype(_i32)

    ve1 = We1.astype(jnp.bfloat16).astype(_f32).T @ att_e1
    ve2 = We2.astype(jnp.bfloat16).astype(_f32).T @ att_e2
    vem = _pad_rows8(jnp.stack([ve1, ve2]))
    attm1 = _pad_rows8(jnp.stack([att_src1, att_dst1]))
    attm2 = _pad_rows8(jnp.stack([att_src2, att_dst2]))
    bw1 = _pad_rows8(b1)
    bw2 = _pad_rows8(b2)
    linm = _pad_rows8(linW[0]).at[1, 0].set(linb[0])
    x2 = jnp.zeros((N2, D_FEAT), _f32).at[:N_NODES].set(x)

    src_t = _tile_edges(src, 0)
    sent = N_NODES + (jnp.arange(EPT - N_EDGES // NW, dtype=_i32) % 16)
    dst_t = _tile_edges(dst, sent)

    ea_pad = jnp.zeros((E2, D_EDGE), _f32).at[:N_EDGES].set(edge_attr)
    ae1, ae2 = _tc_ae(ea_pad, vem)
    ae1 = ae1[:N_EDGES]
    ae2 = ae2[:N_EDGES]
    ae1_t = _tile_edges(ae1, jnp.float32(-1e30))

    h1, asrc1, adst1 = _tc_n1(x2, W1.T, attm1)

    m1 = _lrelu_scalar(
        jnp.max(asrc1[:N_NODES]) + jnp.max(adst1[:N_NODES])
        + jnp.maximum(jnp.max(ae1), 0.0))
    mb1 = jnp.full((8, 128), m1, _f32)

    s1_t, stat1 = _sc_scores(src_t, dst_t, ae1_t, asrc1, adst1, 2)
    stat1 = stat1.reshape(NW, N2, 2)
    p1_t = _tc_exp(s1_t.reshape(E2), mb1).reshape(NW, NG, G)
    acc1, psum1 = _sc_scatter(src_t, dst_t, p1_t, h1)

    h2, asrc2, adst2, degc = _tc_n2(h1, asrc1, adst1, mb1, acc1, stat1,
                                    psum1, bw1, W2.T, attm2)

    m2 = _lrelu_scalar(
        jnp.max(asrc2[:N_NODES]) + jnp.max(adst2[:N_NODES])
        + jnp.maximum(jnp.max(ae2), 0.0))
    mb2 = jnp.full((8, 128), m2, _f32)

    ae2s_t = _tile_edges(ae2, jnp.float32(-1e30))
    s2_t, stat2 = _sc_scores(src_t, dst_t, ae2s_t, asrc2, adst2, 1)
    stat2 = stat2.reshape(NW, N2)
    p2_t = _tc_exp(s2_t.reshape(E2), mb2).reshape(NW, NG, G)
    acc2, psum2 = _sc_scatter(src_t, dst_t, p2_t, h2)

    y = _tc_n3(h2, asrc2, adst2, degc, mb2, acc2, stat2, psum2, bw2, linm)
    return y[:N_NODES].reshape(N_NODES, 1)
